# Initial kernel scaffold; baseline (speedup 1.0000x reference)
import jax
import jax.numpy as jnp
from jax.experimental import pallas as pl


def kernel(input, edge_index, edge_weight_low, edge_weight_high, W_low, W_high, W_mlp, ln_low_g, ln_low_b, ln_high_g, ln_high_b, ln_mlp_g, ln_mlp_b, att_vec_low, att_vec_high, att_vec_mlp, att_vec):
    raise NotImplementedError("write your pallas kernel here")


# speedup vs baseline: 2.4918x; 2.4918x over previous
import functools

import jax
import jax.numpy as jnp
from jax import lax
from jax.experimental import pallas as pl
from jax.experimental.pallas import tpu as pltpu
from jax.experimental.pallas import tpu_sc as plsc

N = 10000
E = 320000
D = 128

NC = 2
NS = 16
L = 16

K = 80
CH = E // K
CPT = CH // NS
SCK = 25
NSC = CPT // SCK
HALF = 5120
NPASS = 2
N_PAD = NPASS * HALF
HPT = HALF // NS

BR = 1000


def _sc_spmm(x, src3, dst3, w3):
    mesh = plsc.VectorSubcoreMesh(core_axis_name="c", subcore_axis_name="s",
                                  num_cores=NC, num_subcores=NS)

    @functools.partial(
        pl.kernel,
        out_type=jax.ShapeDtypeStruct((NC, N_PAD, D), jnp.float32),
        mesh=mesh,
        scratch_types=[
            pltpu.VMEM((SCK, K), jnp.int32),
            pltpu.VMEM((SCK, K), jnp.int32),
            pltpu.VMEM((SCK, K), jnp.float32),
            pltpu.VMEM((K, D), jnp.float32),
            pltpu.VMEM((K,), jnp.int32),
            pltpu.VMEM_SHARED((HALF, D), jnp.float32),
            pltpu.SemaphoreType.DMA,
        ],
    )
    def spmm(x_hbm, src_hbm, dst_hbm, w_hbm, out_hbm,
             src_v, dst_v, w_v, rows_v, dloc_v, acc_sh, sem):
        c = lax.axis_index("c")
        s = lax.axis_index("s")
        row0 = s * HPT
        zv = jnp.zeros((L,), jnp.float32)

        for p in range(NPASS):
            lo = p * HALF

            def zrow(k, carry):
                for q in range(D // L):
                    rows_v[k, pl.ds(q * L, L)] = zv
                return carry

            lax.fori_loop(0, K, zrow, 0)
            for i in range(HPT // K):
                pltpu.sync_copy(rows_v, acc_sh.at[pl.ds(row0 + i * K, K)])
            plsc.subcore_barrier()

            def superchunk(sc_i, carry):
                pltpu.sync_copy(src_hbm.at[s, sc_i], src_v)
                pltpu.sync_copy(dst_hbm.at[s, sc_i], dst_v)
                pltpu.sync_copy(w_hbm.at[c * NS + s, sc_i], w_v)

                def chunk(j, carry1):
                    pltpu.async_copy(x_hbm.at[src_v.at[j]], rows_v, sem).wait()

                    def scale_group(g, carry2):
                        sl16 = pl.ds(g * L, L)
                        d16 = dst_v[j, sl16]
                        ok = (d16 >= lo) & (d16 < lo + HALF)
                        dloc_v[sl16] = jnp.where(ok, d16 - lo, -1)
                        wv = w_v[j, sl16]
                        for jj in range(L):
                            wj = lax.gather(
                                wv, jnp.full((L, 1), jj, jnp.int32),
                                lax.GatherDimensionNumbers(
                                    offset_dims=(), collapsed_slice_dims=(0,),
                                    start_index_map=(0,)),
                                (1,),
                                mode=lax.GatherScatterMode.PROMISE_IN_BOUNDS)
                            k = g * L + jj
                            for q in range(D // L):
                                sl = pl.ds(q * L, L)
                                rows_v[k, sl] = rows_v[k, sl] * wj
                        return carry2

                    lax.fori_loop(0, K // L, scale_group, 0)
                    pltpu.sync_copy(
                        rows_v,
                        acc_sh.at[plsc.Indices(dloc_v, ignored_value=-1)],
                        add=True)
                    return carry1

                lax.fori_loop(0, SCK, chunk, 0)
                return carry

            lax.fori_loop(0, NSC, superchunk, 0)
            plsc.subcore_barrier()
            pltpu.sync_copy(acc_sh.at[pl.ds(row0, HPT)],
                            out_hbm.at[c, pl.ds(lo + row0, HPT)])
            plsc.subcore_barrier()

    return spmm(x, src3, dst3, w3)


def _leaky(v):
    return jnp.where(v >= 0, v, 0.01 * v)


def _ln(v, g, b):
    mu = jnp.mean(v, axis=1, keepdims=True)
    d = v - mu
    var = jnp.mean(d * d, axis=1, keepdims=True)
    return d * lax.rsqrt(var + 1e-5) * g + b


def _tc_body(x_ref, al_ref, ah_ref, wl_ref, wh_ref, wm_ref,
             lg_ref, lb_ref, hg_ref, hb_ref, mg_ref, mb_ref,
             avl_ref, avh_ref, avm_ref, att_ref, o_ref):
    ol = _leaky(jnp.dot(al_ref[0], wl_ref[...], precision="highest"))
    oh = _leaky(jnp.dot(ah_ref[0], wh_ref[...], precision="highest"))
    om = _leaky(jnp.dot(x_ref[...], wm_ref[...], precision="highest"))
    nl = _ln(ol, lg_ref[...], lb_ref[...])
    nh = _ln(oh, hg_ref[...], hb_ref[...])
    nm = _ln(om, mg_ref[...], mb_ref[...])
    s0 = jax.nn.sigmoid(jnp.sum(nl * avl_ref[...], axis=1, keepdims=True))
    s1 = jax.nn.sigmoid(jnp.sum(nh * avh_ref[...], axis=1, keepdims=True))
    s2 = jax.nn.sigmoid(jnp.sum(nm * avm_ref[...], axis=1, keepdims=True))
    logits = [(s0 * att_ref[0, j] + s1 * att_ref[1, j] + s2 * att_ref[2, j]) / 3.0
              for j in range(3)]
    m = jnp.maximum(jnp.maximum(logits[0], logits[1]), logits[2])
    e = [jnp.exp(lj - m) for lj in logits]
    den = e[0] + e[1] + e[2]
    o_ref[...] = 3.0 * ((e[0] / den) * ol + (e[1] / den) * oh + (e[2] / den) * om)


def _tc_post(x, agg, W_low, W_high, W_mlp, lg, lb, hg, hb, mg, mb,
             avl, avh, avm, att):
    grid = N // BR
    big = pl.BlockSpec((BR, D), lambda i: (i, 0))
    low = pl.BlockSpec((1, BR, D), lambda i: (0, i, 0))
    high = pl.BlockSpec((1, BR, D), lambda i: (1, i, 0))
    wspec = pl.BlockSpec((D, D), lambda i: (0, 0))
    vspec = pl.BlockSpec((1, D), lambda i: (0, 0))
    sspec = pl.BlockSpec(memory_space=pltpu.SMEM)
    return pl.pallas_call(
        _tc_body,
        grid=(grid,),
        in_specs=[big, low, high, wspec, wspec, wspec,
                  vspec, vspec, vspec, vspec, vspec, vspec,
                  vspec, vspec, vspec, sspec],
        out_specs=pl.BlockSpec((BR, D), lambda i: (i, 0)),
        out_shape=jax.ShapeDtypeStruct((N, D), jnp.float32),
    )(x, agg, agg, W_low, W_high, W_mlp, lg, lb, hg, hb, mg, mb,
      avl, avh, avm, att)


def kernel(input, edge_index, edge_weight_low, edge_weight_high, W_low, W_high,
           W_mlp, ln_low_g, ln_low_b, ln_high_g, ln_high_b, ln_mlp_g, ln_mlp_b,
           att_vec_low, att_vec_high, att_vec_mlp, att_vec):
    src3 = edge_index[0].reshape(NS, NSC, SCK, K)
    dst3 = edge_index[1].reshape(NS, NSC, SCK, K)
    w3 = jnp.concatenate([edge_weight_low, edge_weight_high]).reshape(
        NC * NS, NSC, SCK, K)
    agg = _sc_spmm(input, src3, dst3, w3)
    return _tc_post(
        input, agg, W_low, W_high, W_mlp,
        ln_low_g.reshape(1, D), ln_low_b.reshape(1, D),
        ln_high_g.reshape(1, D), ln_high_b.reshape(1, D),
        ln_mlp_g.reshape(1, D), ln_mlp_b.reshape(1, D),
        att_vec_low.reshape(1, D), att_vec_high.reshape(1, D),
        att_vec_mlp.reshape(1, D), att_vec)


# speedup vs baseline: 4.0662x; 1.6318x over previous
import functools

import jax
import jax.numpy as jnp
from jax import lax
from jax.experimental import pallas as pl
from jax.experimental.pallas import tpu as pltpu
from jax.experimental.pallas import tpu_sc as plsc

N = 10000
E = 320000
D = 128

NC = 2
NS = 16
L = 16

K = 32
CPT = E // K // NS
SCK = 25
NSC = CPT // SCK
N_PAD = 10112
HPT = N_PAD // NS

BR = 1000


def _sc_spmm(x, src3, dst3, w3):
    mesh = plsc.VectorSubcoreMesh(core_axis_name="c", subcore_axis_name="s",
                                  num_cores=NC, num_subcores=NS)

    @functools.partial(
        pl.kernel,
        out_type=jax.ShapeDtypeStruct((NC, N_PAD, D), jnp.float32),
        mesh=mesh,
        scratch_types=[
            pltpu.VMEM((SCK, K), jnp.int32),
            pltpu.VMEM((SCK, K), jnp.int32),
            pltpu.VMEM((SCK, K), jnp.float32),
            pltpu.VMEM((K, D), jnp.float32),
            pltpu.VMEM((K, D), jnp.float32),
            pltpu.VMEM_SHARED((N_PAD, D), jnp.float32),
            pltpu.SemaphoreType.DMA,
            pltpu.SemaphoreType.DMA,
        ],
    )
    def spmm(x_hbm, src_hbm, dst_hbm, w_hbm, out_hbm,
             src_v, dst_v, w_v, rows0_v, rows1_v, acc_sh, sem0, sem1):
        c = lax.axis_index("c")
        s = lax.axis_index("s")
        row0 = s * HPT
        zv = jnp.zeros((L,), jnp.float32)

        def zrow(k, carry):
            for q in range(D // L):
                rows0_v[k, pl.ds(q * L, L)] = zv
            return carry

        lax.fori_loop(0, K, zrow, 0)
        nfull, rem = divmod(HPT, K)
        for i in range(nfull):
            pltpu.sync_copy(rows0_v, acc_sh.at[pl.ds(row0 + i * K, K)])
        if rem:
            pltpu.sync_copy(rows0_v.at[pl.ds(0, rem)],
                            acc_sh.at[pl.ds(row0 + nfull * K, rem)])
        plsc.subcore_barrier()

        def scale(buf, j):
            for g in range(K // L):
                wv = w_v[j, pl.ds(g * L, L)]
                for jj in range(L):
                    wj = lax.gather(
                        wv, jnp.full((L, 1), jj, jnp.int32),
                        lax.GatherDimensionNumbers(
                            offset_dims=(), collapsed_slice_dims=(0,),
                            start_index_map=(0,)),
                        (1,), mode=lax.GatherScatterMode.PROMISE_IN_BOUNDS)
                    k = g * L + jj
                    for q in range(D // L):
                        sl = pl.ds(q * L, L)
                        buf[k, sl] = buf[k, sl] * wj

        def superchunk(sc_i, carry):
            pltpu.sync_copy(src_hbm.at[s, sc_i], src_v)
            pltpu.sync_copy(dst_hbm.at[s, sc_i], dst_v)
            pltpu.sync_copy(w_hbm.at[c * NS + s, sc_i], w_v)
            pltpu.async_copy(x_hbm.at[src_v.at[0]], rows0_v, sem0)

            def pair(j2, carry1):
                a = 2 * j2
                pltpu.make_async_copy(
                    x_hbm.at[src_v.at[a]], rows0_v, sem0).wait()

                @pl.when(a + 1 < SCK)
                def _():
                    pltpu.async_copy(x_hbm.at[src_v.at[a + 1]], rows1_v, sem1)

                scale(rows0_v, a)
                pltpu.sync_copy(rows0_v, acc_sh.at[dst_v.at[a]], add=True)

                @pl.when(a + 1 < SCK)
                def _():
                    pltpu.make_async_copy(
                        x_hbm.at[src_v.at[a + 1]], rows1_v, sem1).wait()

                    @pl.when(a + 2 < SCK)
                    def _():
                        pltpu.async_copy(x_hbm.at[src_v.at[a + 2]], rows0_v,
                                         sem0)

                    scale(rows1_v, a + 1)
                    pltpu.sync_copy(rows1_v, acc_sh.at[dst_v.at[a + 1]],
                                    add=True)
                return carry1

            lax.fori_loop(0, (SCK + 1) // 2, pair, 0)
            return carry

        lax.fori_loop(0, NSC, superchunk, 0)
        plsc.subcore_barrier()
        pltpu.sync_copy(acc_sh.at[pl.ds(row0, HPT)],
                        out_hbm.at[c, pl.ds(row0, HPT)])

    return spmm(x, src3, dst3, w3)


def _leaky(v):
    return jnp.where(v >= 0, v, 0.01 * v)


def _ln(v, g, b):
    mu = jnp.mean(v, axis=1, keepdims=True)
    d = v - mu
    var = jnp.mean(d * d, axis=1, keepdims=True)
    return d * lax.rsqrt(var + 1e-5) * g + b


def _tc_body(x_ref, al_ref, ah_ref, wl_ref, wh_ref, wm_ref,
             lg_ref, lb_ref, hg_ref, hb_ref, mg_ref, mb_ref,
             avl_ref, avh_ref, avm_ref, att_ref, o_ref):
    ol = _leaky(jnp.dot(al_ref[0], wl_ref[...], precision="highest"))
    oh = _leaky(jnp.dot(ah_ref[0], wh_ref[...], precision="highest"))
    om = _leaky(jnp.dot(x_ref[...], wm_ref[...], precision="highest"))
    nl = _ln(ol, lg_ref[...], lb_ref[...])
    nh = _ln(oh, hg_ref[...], hb_ref[...])
    nm = _ln(om, mg_ref[...], mb_ref[...])
    s0 = jax.nn.sigmoid(jnp.sum(nl * avl_ref[...], axis=1, keepdims=True))
    s1 = jax.nn.sigmoid(jnp.sum(nh * avh_ref[...], axis=1, keepdims=True))
    s2 = jax.nn.sigmoid(jnp.sum(nm * avm_ref[...], axis=1, keepdims=True))
    logits = [(s0 * att_ref[0, j] + s1 * att_ref[1, j] + s2 * att_ref[2, j]) / 3.0
              for j in range(3)]
    m = jnp.maximum(jnp.maximum(logits[0], logits[1]), logits[2])
    e = [jnp.exp(lj - m) for lj in logits]
    den = e[0] + e[1] + e[2]
    o_ref[...] = 3.0 * ((e[0] / den) * ol + (e[1] / den) * oh + (e[2] / den) * om)


def _tc_post(x, agg, W_low, W_high, W_mlp, lg, lb, hg, hb, mg, mb,
             avl, avh, avm, att):
    grid = N // BR
    big = pl.BlockSpec((BR, D), lambda i: (i, 0))
    low = pl.BlockSpec((1, BR, D), lambda i: (0, i, 0))
    high = pl.BlockSpec((1, BR, D), lambda i: (1, i, 0))
    wspec = pl.BlockSpec((D, D), lambda i: (0, 0))
    vspec = pl.BlockSpec((1, D), lambda i: (0, 0))
    sspec = pl.BlockSpec(memory_space=pltpu.SMEM)
    return pl.pallas_call(
        _tc_body,
        grid=(grid,),
        in_specs=[big, low, high, wspec, wspec, wspec,
                  vspec, vspec, vspec, vspec, vspec, vspec,
                  vspec, vspec, vspec, sspec],
        out_specs=pl.BlockSpec((BR, D), lambda i: (i, 0)),
        out_shape=jax.ShapeDtypeStruct((N, D), jnp.float32),
    )(x, agg, agg, W_low, W_high, W_mlp, lg, lb, hg, hb, mg, mb,
      avl, avh, avm, att)


def kernel(input, edge_index, edge_weight_low, edge_weight_high, W_low, W_high,
           W_mlp, ln_low_g, ln_low_b, ln_high_g, ln_high_b, ln_mlp_g, ln_mlp_b,
           att_vec_low, att_vec_high, att_vec_mlp, att_vec):
    src3 = edge_index[0].reshape(NS, NSC, SCK, K)
    dst3 = edge_index[1].reshape(NS, NSC, SCK, K)
    w3 = jnp.concatenate([edge_weight_low, edge_weight_high]).reshape(
        NC * NS, NSC, SCK, K)
    agg = _sc_spmm(input, src3, dst3, w3)
    return _tc_post(
        input, agg, W_low, W_high, W_mlp,
        ln_low_g.reshape(1, D), ln_low_b.reshape(1, D),
        ln_high_g.reshape(1, D), ln_high_b.reshape(1, D),
        ln_mlp_g.reshape(1, D), ln_mlp_b.reshape(1, D),
        att_vec_low.reshape(1, D), att_vec_high.reshape(1, D),
        att_vec_mlp.reshape(1, D), att_vec)
